# baseline (speedup 1.0000x reference)
import jax
import jax.numpy as jnp
from jax.experimental import pallas as pl

BLK = 256

PI32 = 3.1415927410125732
PI_H = 3.1416015625
PI_L = -8.821487426757812e-06
C_CORR = 1.3913767595918975e-08
INV_2PI = 0.15915494309189535

SIN_COEF = (6.282113075256348, -41.20405960083008, 78.80842590332031,
            -58.086524963378906)
COS_COEF = (0.9985257983207703, -19.549816131591797, 61.076332092285156,
            -59.49797058105469)


def _frac(x):
    return x - jnp.round(x)


def _body(a_ref, low_ref, scale_ref, pows_ref, idx_ref, o_ref):
    a = a_ref[...]
    an = jnp.clip((a - low_ref[...]) * scale_ref[...] + 1.0, -1.0, 1.0)
    AF = pows_ref.shape[-1]
    idx = jnp.broadcast_to(idx_ref[...], (an.shape[0], AF))
    a_exp = jnp.take_along_axis(an, idx, axis=1)

    s = a_exp * pows_ref[...]
    v = s * jnp.float32(PI32)
    tt = s * jnp.float32(4097.0)
    s_h = tt - (tt - s)
    s_l = s - s_h
    e = s_h * jnp.float32(PI_H) - v
    e = e + s_h * jnp.float32(PI_L)
    e = e + s_l * jnp.float32(PI_H)
    e = e + s_l * jnp.float32(PI_L)

    r = _frac(0.5 * s) + (s * jnp.float32(C_CORR) - e * jnp.float32(INV_2PI))
    r = _frac(r)

    u = r * r
    ps = jnp.float32(SIN_COEF[-1])
    for cf in SIN_COEF[-2::-1]:
        ps = ps * u + jnp.float32(cf)
    pc = jnp.float32(COS_COEF[-1])
    for cf in COS_COEF[-2::-1]:
        pc = pc * u + jnp.float32(cf)
    o_ref[:, :AF] = r * ps
    o_ref[:, AF:] = pc


def kernel(action, action_low, action_high, freqs):
    B, A = action.shape
    F = freqs.shape[0]
    scale = (2.0 / (action_high - action_low)).reshape(1, A)
    low = action_low.reshape(1, A)
    pows = jax.lax.bitcast_convert_type(
        (jnp.arange(F, dtype=jnp.int32) + 127) << 23, jnp.float32)
    pows_tiled = jnp.tile(pows, (A,)).reshape(1, A * F)
    idx = (jnp.arange(A * F, dtype=jnp.int32) // F).reshape(1, A * F)
    out = pl.pallas_call(
        _body,
        grid=(B // BLK,),
        in_specs=[
            pl.BlockSpec((BLK, A), lambda i: (i, 0)),
            pl.BlockSpec((1, A), lambda i: (0, 0)),
            pl.BlockSpec((1, A), lambda i: (0, 0)),
            pl.BlockSpec((1, A * F), lambda i: (0, 0)),
            pl.BlockSpec((1, A * F), lambda i: (0, 0)),
        ],
        out_specs=pl.BlockSpec((BLK, 2 * A * F), lambda i: (i, 0)),
        out_shape=jax.ShapeDtypeStruct((B, 2 * A * F), jnp.float32),
    )(action, low, scale, pows_tiled, idx)
    return out


# speedup vs baseline: 1.0106x; 1.0106x over previous
import jax
import jax.numpy as jnp
from jax.experimental import pallas as pl

BLK = 512

PI32 = 3.1415927410125732
PI_H = 3.1416015625
PI_L = -8.821487426757812e-06
C_CORR = 1.3913767595918975e-08
INV_2PI = 0.15915494309189535

SIN_COEF = (6.282113075256348, -41.20405960083008, 78.80842590332031,
            -58.086524963378906)
COS_COEF = (0.9985257983207703, -19.549816131591797, 61.076332092285156,
            -59.49797058105469)


def _frac(x):
    return x - jnp.round(x)


def _body(a_ref, low_ref, scale_ref, pows_ref, idx_ref, o_ref):
    a = a_ref[...]
    an = jnp.clip((a - low_ref[...]) * scale_ref[...] + 1.0, -1.0, 1.0)
    AF = pows_ref.shape[-1]
    idx = jnp.broadcast_to(idx_ref[...], (an.shape[0], AF))
    a_exp = jnp.take_along_axis(an, idx, axis=1)

    s = a_exp * pows_ref[...]
    v = s * jnp.float32(PI32)
    tt = s * jnp.float32(4097.0)
    s_h = tt - (tt - s)
    s_l = s - s_h
    e = s_h * jnp.float32(PI_H) - v
    e = e + s_h * jnp.float32(PI_L)
    e = e + s_l * jnp.float32(PI_H)
    e = e + s_l * jnp.float32(PI_L)

    r = _frac(0.5 * s) + (s * jnp.float32(C_CORR) - e * jnp.float32(INV_2PI))
    r = _frac(r)

    u = r * r
    ps = jnp.float32(SIN_COEF[-1])
    for cf in SIN_COEF[-2::-1]:
        ps = ps * u + jnp.float32(cf)
    pc = jnp.float32(COS_COEF[-1])
    for cf in COS_COEF[-2::-1]:
        pc = pc * u + jnp.float32(cf)
    o_ref[:, :AF] = r * ps
    o_ref[:, AF:] = pc


def kernel(action, action_low, action_high, freqs):
    B, A = action.shape
    F = freqs.shape[0]
    scale = (2.0 / (action_high - action_low)).reshape(1, A)
    low = action_low.reshape(1, A)
    pows = jax.lax.bitcast_convert_type(
        (jnp.arange(F, dtype=jnp.int32) + 127) << 23, jnp.float32)
    pows_tiled = jnp.tile(pows, (A,)).reshape(1, A * F)
    idx = (jnp.arange(A * F, dtype=jnp.int32) // F).reshape(1, A * F)
    out = pl.pallas_call(
        _body,
        grid=(B // BLK,),
        in_specs=[
            pl.BlockSpec((BLK, A), lambda i: (i, 0)),
            pl.BlockSpec((1, A), lambda i: (0, 0)),
            pl.BlockSpec((1, A), lambda i: (0, 0)),
            pl.BlockSpec((1, A * F), lambda i: (0, 0)),
            pl.BlockSpec((1, A * F), lambda i: (0, 0)),
        ],
        out_specs=pl.BlockSpec((BLK, 2 * A * F), lambda i: (i, 0)),
        out_shape=jax.ShapeDtypeStruct((B, 2 * A * F), jnp.float32),
    )(action, low, scale, pows_tiled, idx)
    return out
